# initial kernel scaffold (unmeasured)
import jax
import jax.numpy as jnp
from jax import lax
from jax.experimental import pallas as pl
from jax.experimental.pallas import tpu as pltpu

T, D, V = 2048, 4096, 16384
TX = T // 2
VY = V // 2
H = VY // 2


def _ag_body(e_ref, s_ref, out_ref, s_recv,
             s_snd, s_rcv, ax_snd, ax_rcv, ay_snd, ay_rcv,
             bx_snd, bx_rcv, by_snd, by_rcv):
    xi = lax.axis_index("x")
    yi = lax.axis_index("y")
    xo = 1 - xi
    yo = 1 - yi
    my_r = xi * TX
    my_c = yi * VY

    barrier = pltpu.get_barrier_semaphore()
    pl.semaphore_signal(barrier, inc=1, device_id=(xo, yi),
                        device_id_type=pl.DeviceIdType.MESH)
    pl.semaphore_signal(barrier, inc=1, device_id=(xi, yo),
                        device_id_type=pl.DeviceIdType.MESH)
    pl.semaphore_wait(barrier, 2)

    s_rdma = pltpu.make_async_remote_copy(
        src_ref=s_ref, dst_ref=s_recv, send_sem=s_snd, recv_sem=s_rcv,
        device_id=(xi, yo), device_id_type=pl.DeviceIdType.MESH)
    s_rdma.start()
    s_rdma.wait()

    inv = 1.0 / (s_ref[:, 0:1] + s_recv[:, 0:1])
    out_ref[pl.ds(my_r, TX), pl.ds(my_c, VY)] = (
        e_ref[...].astype(jnp.float32) * inv).astype(jnp.bfloat16)

    ax = pltpu.make_async_remote_copy(
        src_ref=out_ref.at[pl.ds(my_r, TX), pl.ds(my_c, VY)],
        dst_ref=out_ref.at[pl.ds(my_r, TX), pl.ds(my_c, VY)],
        send_sem=ax_snd, recv_sem=ax_rcv,
        device_id=(xo, yi), device_id_type=pl.DeviceIdType.MESH)
    ay = pltpu.make_async_remote_copy(
        src_ref=out_ref.at[pl.ds(my_r, TX), pl.ds(my_c, VY)],
        dst_ref=out_ref.at[pl.ds(my_r, TX), pl.ds(my_c, VY)],
        send_sem=ay_snd, recv_sem=ay_rcv,
        device_id=(xi, yo), device_id_type=pl.DeviceIdType.MESH)
    ax.start()
    ay.start()

    ay.wait_recv()
    bx = pltpu.make_async_remote_copy(
        src_ref=out_ref.at[pl.ds(my_r, TX), pl.ds(yo * VY, H)],
        dst_ref=out_ref.at[pl.ds(my_r, TX), pl.ds(yo * VY, H)],
        send_sem=bx_snd, recv_sem=bx_rcv,
        device_id=(xo, yi), device_id_type=pl.DeviceIdType.MESH)
    bx.start()

    ax.wait_recv()
    by = pltpu.make_async_remote_copy(
        src_ref=out_ref.at[pl.ds(xo * TX, TX), pl.ds(my_c + H, H)],
        dst_ref=out_ref.at[pl.ds(xo * TX, TX), pl.ds(my_c + H, H)],
        send_sem=by_snd, recv_sem=by_rcv,
        device_id=(xi, yo), device_id_type=pl.DeviceIdType.MESH)
    by.start()

    ax.wait_send()
    ay.wait_send()
    bx.wait()
    by.wait()


def kernel(x, W):
    xi = lax.axis_index("x")
    x_half = lax.dynamic_slice(x, (xi * TX, 0), (TX, D))
    logits = jnp.dot(x_half.astype(jnp.bfloat16), W.astype(jnp.bfloat16),
                     preferred_element_type=jnp.float32)
    e = jnp.exp(logits)
    s = jnp.sum(e, axis=1, keepdims=True)
    s_pad = jnp.broadcast_to(s, (TX, 128))
    e_bf = e.astype(jnp.bfloat16)

    return pl.pallas_call(
        _ag_body,
        out_shape=jax.ShapeDtypeStruct((T, V), jnp.bfloat16),
        in_specs=[pl.BlockSpec(memory_space=pltpu.VMEM),
                  pl.BlockSpec(memory_space=pltpu.VMEM)],
        out_specs=pl.BlockSpec(memory_space=pltpu.VMEM),
        scratch_shapes=[pltpu.VMEM((TX, 128), jnp.float32)]
        + [pltpu.SemaphoreType.DMA] * 10,
        compiler_params=pltpu.CompilerParams(collective_id=0),
    )(e_bf, s_pad)


# baseline (device time: 449358 ns/iter reference)
import jax
import jax.numpy as jnp
from jax import lax
from jax.experimental import pallas as pl
from jax.experimental.pallas import tpu as pltpu

T, D, V = 2048, 4096, 16384
TX = T // 2
VY = V // 2
H = VY // 2
NCHUNK = 2048


def _ag_body(e_ref, s_ref, out_ref, p_ref, s_recv,
             loc_sem, s_snd, s_rcv, ax_snd, ax_rcv, ay_snd, ay_rcv,
             bx_snd, bx_rcv, by_snd, by_rcv):
    xi = lax.axis_index("x")
    yi = lax.axis_index("y")
    xo = 1 - xi
    yo = 1 - yi
    my_r = xi * TX
    my_c = yi * VY

    barrier = pltpu.get_barrier_semaphore()
    pl.semaphore_signal(barrier, inc=1, device_id=(xo, yi),
                        device_id_type=pl.DeviceIdType.MESH)
    pl.semaphore_signal(barrier, inc=1, device_id=(xi, yo),
                        device_id_type=pl.DeviceIdType.MESH)
    pl.semaphore_wait(barrier, 2)

    s_rdma = pltpu.make_async_remote_copy(
        src_ref=s_ref, dst_ref=s_recv, send_sem=s_snd, recv_sem=s_rcv,
        device_id=(xi, yo), device_id_type=pl.DeviceIdType.MESH)
    s_rdma.start()
    s_rdma.wait()

    inv = 1.0 / (s_ref[:, 0:1] + s_recv[:, 0:1])
    for c0 in range(0, VY, NCHUNK):
        p_ref[:, c0:c0 + NCHUNK] = (
            e_ref[:, c0:c0 + NCHUNK].astype(jnp.float32) * inv
        ).astype(jnp.bfloat16)

    loc = pltpu.make_async_copy(
        p_ref, out_ref.at[pl.ds(my_r, TX), pl.ds(my_c, VY)], loc_sem)
    loc.start()
    ax = pltpu.make_async_remote_copy(
        src_ref=p_ref,
        dst_ref=out_ref.at[pl.ds(my_r, TX), pl.ds(my_c, VY)],
        send_sem=ax_snd, recv_sem=ax_rcv,
        device_id=(xo, yi), device_id_type=pl.DeviceIdType.MESH)
    ay = pltpu.make_async_remote_copy(
        src_ref=p_ref,
        dst_ref=out_ref.at[pl.ds(my_r, TX), pl.ds(my_c, VY)],
        send_sem=ay_snd, recv_sem=ay_rcv,
        device_id=(xi, yo), device_id_type=pl.DeviceIdType.MESH)
    ax.start()
    ay.start()

    ay.wait_recv()
    bx = pltpu.make_async_remote_copy(
        src_ref=out_ref.at[pl.ds(my_r, TX), pl.ds(yo * VY, H)],
        dst_ref=out_ref.at[pl.ds(my_r, TX), pl.ds(yo * VY, H)],
        send_sem=bx_snd, recv_sem=bx_rcv,
        device_id=(xo, yi), device_id_type=pl.DeviceIdType.MESH)
    bx.start()

    ax.wait_recv()
    by = pltpu.make_async_remote_copy(
        src_ref=out_ref.at[pl.ds(xo * TX, TX), pl.ds(my_c + H, H)],
        dst_ref=out_ref.at[pl.ds(xo * TX, TX), pl.ds(my_c + H, H)],
        send_sem=by_snd, recv_sem=by_rcv,
        device_id=(xi, yo), device_id_type=pl.DeviceIdType.MESH)
    by.start()

    loc.wait()
    ax.wait_send()
    ay.wait_send()
    bx.wait()
    by.wait()


def kernel(x, W):
    xi = lax.axis_index("x")
    x_half = lax.dynamic_slice(x, (xi * TX, 0), (TX, D))
    logits = jnp.dot(x_half.astype(jnp.bfloat16), W.astype(jnp.bfloat16),
                     preferred_element_type=jnp.float32)
    e = jnp.exp(logits)
    s = jnp.sum(e, axis=1, keepdims=True)
    s_pad = jnp.broadcast_to(s, (TX, 128))
    e_bf = e.astype(jnp.bfloat16)

    return pl.pallas_call(
        _ag_body,
        out_shape=jax.ShapeDtypeStruct((T, V), jnp.bfloat16),
        in_specs=[pl.BlockSpec(memory_space=pltpu.VMEM),
                  pl.BlockSpec(memory_space=pltpu.VMEM)],
        out_specs=pl.BlockSpec(memory_space=pl.ANY),
        scratch_shapes=[pltpu.VMEM((TX, VY), jnp.bfloat16),
                        pltpu.VMEM((TX, 128), jnp.float32)]
        + [pltpu.SemaphoreType.DMA] * 11,
        compiler_params=pltpu.CompilerParams(collective_id=0),
    )(e_bf, s_pad)


# device time: 382581 ns/iter; 1.1745x vs baseline; 1.1745x over previous
import jax
import jax.numpy as jnp
from jax import lax
from jax.experimental import pallas as pl
from jax.experimental.pallas import tpu as pltpu

T, D, V = 2048, 4096, 16384
TX = T // 2
VY = V // 2
H = VY // 2
KT = 512
NSUB = VY // KT
CT = 1024
NCT = VY // CT
NBUF = 4

_MESH = pl.DeviceIdType.MESH


def _body(x_ref, w_hbm, out_ref,
          wbuf, e_ref, nbuf, sacc, srcv, stot, stot_o,
          wsem, lin, lout,
          s1_snd, s1_rcv, s2_snd, s2_rcv,
          ax_snd, ax_rcv, ay_snd, ay_rcv,
          bx_snd, bx_rcv, by_snd, by_rcv):
    xi = lax.axis_index("x")
    yi = lax.axis_index("y")
    xo = 1 - xi
    yo = 1 - yi
    my_r = xi * TX
    my_c = yi * VY

    barrier = pltpu.get_barrier_semaphore()
    pl.semaphore_signal(barrier, inc=1, device_id=(xo, yi),
                        device_id_type=_MESH)
    pl.semaphore_signal(barrier, inc=1, device_id=(xi, yo),
                        device_id_type=_MESH)
    pl.semaphore_wait(barrier, 2)

    wdma = [None] * NSUB

    def w_start(i):
        d = pltpu.make_async_copy(
            w_hbm.at[:, pl.ds(i * KT, KT)], wbuf.at[i % 2], wsem.at[i % 2])
        d.start()
        wdma[i] = d

    w_start(0)
    w_start(1)

    sacc[:, :] = jnp.zeros((TX, 128), jnp.float32)
    ax = [None] * NCT
    ay = [None] * NCT
    for i in range(NSUB):
        wdma[i].wait()
        wb = wbuf[i % 2, :, :].astype(jnp.bfloat16)
        l = lax.dot_general(x_ref[...], wb, (((1,), (0,)), ((), ())),
                            preferred_element_type=jnp.float32)
        e = jnp.exp(l)
        sacc[:, 0:1] = sacc[:, 0:1] + jnp.sum(e, axis=1, keepdims=True)
        e_ref[:, i * KT:(i + 1) * KT] = e.astype(jnp.bfloat16)
        if i + 2 < NSUB:
            w_start(i + 2)
        if i % 2 == 1:
            ct = i // 2
            ax[ct] = pltpu.make_async_remote_copy(
                src_ref=e_ref.at[:, pl.ds(ct * CT, CT)],
                dst_ref=out_ref.at[pl.ds(my_r, TX), pl.ds(my_c + ct * CT, CT)],
                send_sem=ax_snd.at[ct], recv_sem=ax_rcv.at[ct],
                device_id=(xo, yi), device_id_type=_MESH)
            ay[ct] = pltpu.make_async_remote_copy(
                src_ref=e_ref.at[:, pl.ds(ct * CT, CT)],
                dst_ref=out_ref.at[pl.ds(my_r, TX), pl.ds(my_c + ct * CT, CT)],
                send_sem=ay_snd.at[ct], recv_sem=ay_rcv.at[ct],
                device_id=(xi, yo), device_id_type=_MESH)
            ax[ct].start()
            ay[ct].start()

    s1 = pltpu.make_async_remote_copy(
        src_ref=sacc, dst_ref=srcv, send_sem=s1_snd, recv_sem=s1_rcv,
        device_id=(xi, yo), device_id_type=_MESH)
    s1.start()
    s1.wait()
    stot[:, :] = sacc[:, :] + srcv[:, :]
    s2 = pltpu.make_async_remote_copy(
        src_ref=stot, dst_ref=stot_o, send_sem=s2_snd, recv_sem=s2_rcv,
        device_id=(xo, yi), device_id_type=_MESH)
    s2.start()
    s2.wait()
    inv_my = 1.0 / stot[:, 0:1]
    inv_o = 1.0 / stot_o[:, 0:1]

    pend = [None] * NBUF
    slot_ctr = [0]

    def alloc():
        s = slot_ctr[0] % NBUF
        slot_ctr[0] += 1
        if pend[s] is not None:
            pend[s].wait()
            pend[s] = None
        return s

    def store_out(s, br, bc, ct):
        d = pltpu.make_async_copy(
            nbuf.at[s],
            out_ref.at[pl.ds(br, TX), pl.ds(bc + ct * CT, CT)],
            lout.at[s])
        d.start()
        pend[s] = d

    def norm_foreign(br, bc, ct, inv):
        s = alloc()
        din = pltpu.make_async_copy(
            out_ref.at[pl.ds(br, TX), pl.ds(bc + ct * CT, CT)],
            nbuf.at[s], lin.at[s])
        din.start()
        din.wait()
        nbuf[s, :, :] = (nbuf[s, :, :].astype(jnp.float32)
                         * inv).astype(jnp.bfloat16)
        store_out(s, br, bc, ct)

    def norm_own(ct):
        s = alloc()
        nbuf[s, :, :] = (e_ref[:, ct * CT:(ct + 1) * CT].astype(jnp.float32)
                         * inv_my).astype(jnp.bfloat16)
        store_out(s, my_r, my_c, ct)

    def drain_pend():
        for s in range(NBUF):
            if pend[s] is not None:
                pend[s].wait()
                pend[s] = None

    for ct in range(4):
        ay[ct].wait_recv()
        norm_foreign(my_r, yo * VY, ct, inv_my)
    drain_pend()
    bx = pltpu.make_async_remote_copy(
        src_ref=out_ref.at[pl.ds(my_r, TX), pl.ds(yo * VY, H)],
        dst_ref=out_ref.at[pl.ds(my_r, TX), pl.ds(yo * VY, H)],
        send_sem=bx_snd, recv_sem=bx_rcv,
        device_id=(xo, yi), device_id_type=_MESH)
    bx.start()

    for ct in range(4, 8):
        ax[ct].wait_recv()
        norm_foreign(xo * TX, my_c, ct, inv_o)
    drain_pend()
    by = pltpu.make_async_remote_copy(
        src_ref=out_ref.at[pl.ds(xo * TX, TX), pl.ds(my_c + H, H)],
        dst_ref=out_ref.at[pl.ds(xo * TX, TX), pl.ds(my_c + H, H)],
        send_sem=by_snd, recv_sem=by_rcv,
        device_id=(xi, yo), device_id_type=_MESH)
    by.start()

    for ct in range(4, 8):
        ay[ct].wait_recv()
        norm_foreign(my_r, yo * VY, ct, inv_my)
    for ct in range(4):
        ax[ct].wait_recv()
        norm_foreign(xo * TX, my_c, ct, inv_o)
    for ct in range(NCT):
        norm_own(ct)
    drain_pend()

    bx.wait_recv()
    by.wait_recv()
    bx.wait_send()
    by.wait_send()
    for ct in range(NCT):
        ax[ct].wait_send()
        ay[ct].wait_send()


def kernel(x, W):
    xi = lax.axis_index("x")
    x_half = lax.dynamic_slice(x, (xi * TX, 0), (TX, D)).astype(jnp.bfloat16)

    return pl.pallas_call(
        _body,
        out_shape=jax.ShapeDtypeStruct((T, V), jnp.bfloat16),
        in_specs=[pl.BlockSpec(memory_space=pltpu.VMEM),
                  pl.BlockSpec(memory_space=pl.ANY)],
        out_specs=pl.BlockSpec(memory_space=pl.ANY),
        scratch_shapes=[
            pltpu.VMEM((2, D, KT), jnp.float32),
            pltpu.VMEM((TX, VY), jnp.bfloat16),
            pltpu.VMEM((NBUF, TX, CT), jnp.bfloat16),
            pltpu.VMEM((TX, 128), jnp.float32),
            pltpu.VMEM((TX, 128), jnp.float32),
            pltpu.VMEM((TX, 128), jnp.float32),
            pltpu.VMEM((TX, 128), jnp.float32),
            pltpu.SemaphoreType.DMA((2,)),
            pltpu.SemaphoreType.DMA((NBUF,)),
            pltpu.SemaphoreType.DMA((NBUF,)),
            pltpu.SemaphoreType.DMA,
            pltpu.SemaphoreType.DMA,
            pltpu.SemaphoreType.DMA,
            pltpu.SemaphoreType.DMA,
            pltpu.SemaphoreType.DMA((NCT,)),
            pltpu.SemaphoreType.DMA((NCT,)),
            pltpu.SemaphoreType.DMA((NCT,)),
            pltpu.SemaphoreType.DMA((NCT,)),
            pltpu.SemaphoreType.DMA,
            pltpu.SemaphoreType.DMA,
            pltpu.SemaphoreType.DMA,
            pltpu.SemaphoreType.DMA,
        ],
        compiler_params=pltpu.CompilerParams(
            collective_id=0, vmem_limit_bytes=100 * 1024 * 1024),
    )(x_half, W)


# device time: 363020 ns/iter; 1.2378x vs baseline; 1.0539x over previous
import jax
import jax.numpy as jnp
from jax import lax
from jax.experimental import pallas as pl
from jax.experimental.pallas import tpu as pltpu

T, D, V = 2048, 4096, 16384
TX = T // 2
VY = V // 2
H = VY // 2
KT = 512
NSUB = VY // KT
CT = 1024
NCT = VY // CT
NFWD = H // CT
NBUF = 4

_MESH = pl.DeviceIdType.MESH


def _body(x_ref, w_hbm, out_ref,
          wbuf, e_ref, nbuf, sacc, srcv_y, srcv_x, srcv_d,
          wsem, lin, lout,
          s1_snd, s1_rcv, s2_snd, s2_rcv, s3_snd, s3_rcv,
          ax_snd, ax_rcv, ay_snd, ay_rcv,
          bx_snd, bx_rcv, by_snd, by_rcv):
    xi = lax.axis_index("x")
    yi = lax.axis_index("y")
    xo = 1 - xi
    yo = 1 - yi
    my_r = xi * TX
    my_c = yi * VY
    ot_r = xo * TX
    ot_c = yo * VY

    barrier = pltpu.get_barrier_semaphore()
    pl.semaphore_signal(barrier, inc=1, device_id=(xo, yi),
                        device_id_type=_MESH)
    pl.semaphore_signal(barrier, inc=1, device_id=(xi, yo),
                        device_id_type=_MESH)
    pl.semaphore_wait(barrier, 2)

    wdma = [None] * NSUB

    def w_start(i):
        d = pltpu.make_async_copy(
            w_hbm.at[:, pl.ds(i * KT, KT)], wbuf.at[i % 2], wsem.at[i % 2])
        d.start()
        wdma[i] = d

    w_start(0)
    w_start(1)

    sacc[:, :] = jnp.zeros((TX, 128), jnp.float32)
    ax = [None] * NCT
    ay = [None] * NCT
    for i in range(NSUB):
        wdma[i].wait()
        wb = wbuf[i % 2, :, :].astype(jnp.bfloat16)
        l = lax.dot_general(x_ref[...], wb, (((1,), (0,)), ((), ())),
                            preferred_element_type=jnp.float32)
        e = jnp.exp(l)
        sacc[:, 0:1] = sacc[:, 0:1] + jnp.sum(e, axis=1, keepdims=True)
        e_ref[:, i * KT:(i + 1) * KT] = e.astype(jnp.bfloat16)
        if i + 2 < NSUB:
            w_start(i + 2)
        if i % 2 == 1:
            ct = i // 2
            ax[ct] = pltpu.make_async_remote_copy(
                src_ref=e_ref.at[:, pl.ds(ct * CT, CT)],
                dst_ref=out_ref.at[pl.ds(my_r, TX), pl.ds(my_c + ct * CT, CT)],
                send_sem=ax_snd.at[ct], recv_sem=ax_rcv.at[ct],
                device_id=(xo, yi), device_id_type=_MESH)
            ay[ct] = pltpu.make_async_remote_copy(
                src_ref=e_ref.at[:, pl.ds(ct * CT, CT)],
                dst_ref=out_ref.at[pl.ds(my_r, TX), pl.ds(my_c + ct * CT, CT)],
                send_sem=ay_snd.at[ct], recv_sem=ay_rcv.at[ct],
                device_id=(xi, yo), device_id_type=_MESH)
            ax[ct].start()
            ay[ct].start()

    s1 = pltpu.make_async_remote_copy(
        src_ref=sacc, dst_ref=srcv_y, send_sem=s1_snd, recv_sem=s1_rcv,
        device_id=(xi, yo), device_id_type=_MESH)
    s1.start()
    s2 = pltpu.make_async_remote_copy(
        src_ref=sacc, dst_ref=srcv_x, send_sem=s2_snd, recv_sem=s2_rcv,
        device_id=(xo, yi), device_id_type=_MESH)
    s2.start()

    bx = [None] * NFWD
    for t in range(NFWD):
        ay[t].wait_recv()
        bx[t] = pltpu.make_async_remote_copy(
            src_ref=out_ref.at[pl.ds(my_r, TX), pl.ds(ot_c + t * CT, CT)],
            dst_ref=out_ref.at[pl.ds(my_r, TX), pl.ds(ot_c + t * CT, CT)],
            send_sem=bx_snd.at[t], recv_sem=bx_rcv.at[t],
            device_id=(xo, yi), device_id_type=_MESH)
        bx[t].start()

    s1.wait_recv()
    s2.wait_recv()
    s3 = pltpu.make_async_remote_copy(
        src_ref=srcv_x, dst_ref=srcv_d, send_sem=s3_snd, recv_sem=s3_rcv,
        device_id=(xi, yo), device_id_type=_MESH)
    s3.start()

    by = [None] * NFWD
    for t in range(NFWD):
        ax[NFWD + t].wait_recv()
        by[t] = pltpu.make_async_remote_copy(
            src_ref=out_ref.at[pl.ds(ot_r, TX), pl.ds(my_c + H + t * CT, CT)],
            dst_ref=out_ref.at[pl.ds(ot_r, TX), pl.ds(my_c + H + t * CT, CT)],
            send_sem=by_snd.at[t], recv_sem=by_rcv.at[t],
            device_id=(xi, yo), device_id_type=_MESH)
        by[t].start()

    s3.wait_recv()
    inv_my = 1.0 / (sacc[:, 0:1] + srcv_y[:, 0:1])
    inv_o = 1.0 / (srcv_x[:, 0:1] + srcv_d[:, 0:1])

    tiles = []
    for t in range(NFWD, NCT):
        tiles.append((my_r, ot_c + t * CT, inv_my,
                      [ay[t].wait_recv], None))
    for t in range(NCT):
        tiles.append((my_r, my_c + t * CT, inv_my, [], t * CT))
    for t in range(NFWD):
        tiles.append((ot_r, my_c + t * CT, inv_o,
                      [ax[t].wait_recv], None))
    for t in range(NFWD):
        tiles.append((my_r, ot_c + t * CT, inv_my,
                      [bx[t].wait_send], None))
        tiles.append((ot_r, my_c + H + t * CT, inv_o,
                      [by[t].wait_send], None))
        tiles.append((ot_r, ot_c + t * CT, inv_o,
                      [bx[t].wait_recv], None))
        tiles.append((ot_r, ot_c + H + t * CT, inv_o,
                      [by[t].wait_recv], None))

    pend = [None] * NBUF
    dins = [None] * len(tiles)

    def issue(j):
        r0, c0, _, waits, e_col = tiles[j]
        for w in waits:
            w()
        s = j % NBUF
        if pend[s] is not None:
            pend[s].wait()
            pend[s] = None
        if e_col is None:
            d = pltpu.make_async_copy(
                out_ref.at[pl.ds(r0, TX), pl.ds(c0, CT)],
                nbuf.at[s], lin.at[s])
            d.start()
            dins[j] = d

    DEPTH = 3
    for j in range(min(DEPTH, len(tiles))):
        issue(j)
    for j, (r0, c0, inv, _, e_col) in enumerate(tiles):
        s = j % NBUF
        if e_col is None:
            dins[j].wait()
            nbuf[s, :, :] = (nbuf[s, :, :].astype(jnp.float32)
                             * inv).astype(jnp.bfloat16)
        else:
            nbuf[s, :, :] = (e_ref[:, e_col:e_col + CT].astype(jnp.float32)
                             * inv).astype(jnp.bfloat16)
        dout = pltpu.make_async_copy(
            nbuf.at[s],
            out_ref.at[pl.ds(r0, TX), pl.ds(c0, CT)],
            lout.at[s])
        dout.start()
        pend[s] = dout
        if j + DEPTH < len(tiles):
            issue(j + DEPTH)

    for s in range(NBUF):
        if pend[s] is not None:
            pend[s].wait()

    s1.wait_send()
    s2.wait_send()
    s3.wait_send()
    for ct in range(NCT):
        ax[ct].wait_send()
        ay[ct].wait_send()


def kernel(x, W):
    xi = lax.axis_index("x")
    x_half = lax.dynamic_slice(x, (xi * TX, 0), (TX, D)).astype(jnp.bfloat16)

    return pl.pallas_call(
        _body,
        out_shape=jax.ShapeDtypeStruct((T, V), jnp.bfloat16),
        in_specs=[pl.BlockSpec(memory_space=pltpu.VMEM),
                  pl.BlockSpec(memory_space=pl.ANY)],
        out_specs=pl.BlockSpec(memory_space=pl.ANY),
        scratch_shapes=[
            pltpu.VMEM((2, D, KT), jnp.float32),
            pltpu.VMEM((TX, VY), jnp.bfloat16),
            pltpu.VMEM((NBUF, TX, CT), jnp.bfloat16),
            pltpu.VMEM((TX, 128), jnp.float32),
            pltpu.VMEM((TX, 128), jnp.float32),
            pltpu.VMEM((TX, 128), jnp.float32),
            pltpu.VMEM((TX, 128), jnp.float32),
            pltpu.SemaphoreType.DMA((2,)),
            pltpu.SemaphoreType.DMA((NBUF,)),
            pltpu.SemaphoreType.DMA((NBUF,)),
            pltpu.SemaphoreType.DMA,
            pltpu.SemaphoreType.DMA,
            pltpu.SemaphoreType.DMA,
            pltpu.SemaphoreType.DMA,
            pltpu.SemaphoreType.DMA,
            pltpu.SemaphoreType.DMA,
            pltpu.SemaphoreType.DMA((NCT,)),
            pltpu.SemaphoreType.DMA((NCT,)),
            pltpu.SemaphoreType.DMA((NCT,)),
            pltpu.SemaphoreType.DMA((NCT,)),
            pltpu.SemaphoreType.DMA((NFWD,)),
            pltpu.SemaphoreType.DMA((NFWD,)),
            pltpu.SemaphoreType.DMA((NFWD,)),
            pltpu.SemaphoreType.DMA((NFWD,)),
        ],
        compiler_params=pltpu.CompilerParams(
            collective_id=0, vmem_limit_bytes=100 * 1024 * 1024),
    )(x_half, W)
